# baseline (speedup 1.0000x reference)
import functools

import jax
import jax.numpy as jnp
from jax import lax
from jax.experimental import pallas as pl
from jax.experimental.pallas import tpu as pltpu
from jax.experimental.pallas import tpu_sc as plsc

_NUM_CORES = 2
_NUM_SUBCORES = 16
_NUM_WORKERS = _NUM_CORES * _NUM_SUBCORES
_IDX_CHUNK = 128
_LANES = 16


@jax.jit
def kernel(colony_ids, embedding):
    B = colony_ids.shape[0]
    V, D = embedding.shape
    rps = 128 // D
    table4 = embedding.reshape(V // rps, 128)
    b_per_w = B // _NUM_WORKERS
    n_chunks = b_per_w // _IDX_CHUNK

    mesh = plsc.VectorSubcoreMesh(core_axis_name="c", subcore_axis_name="s")

    @functools.partial(
        pl.kernel,
        mesh=mesh,
        out_type=jax.ShapeDtypeStruct((B * D,), jnp.float32),
        scratch_types=[
            pltpu.VMEM((b_per_w,), jnp.int32),
            pltpu.VMEM((b_per_w,), jnp.int32),
            pltpu.VMEM((b_per_w, 128), jnp.float32),
            pltpu.VMEM((b_per_w * D,), jnp.float32),
            pltpu.SemaphoreType.DMA,
        ],
        compiler_params=pltpu.CompilerParams(
            use_tc_tiling_on_sc=True, needs_layout_passes=False
        ),
    )
    def _gather(table_hbm, idx_hbm, out_hbm, idx_v, idx4_v, rows4_v, out_v, sem):
        wid = lax.axis_index("s") * _NUM_CORES + lax.axis_index("c")
        base = wid * b_per_w
        pltpu.sync_copy(idx_hbm.at[pl.ds(base, b_per_w)], idx_v)

        def shift_body(t, carry):
            sl = pl.ds(t * _LANES, _LANES)
            idx4_v[sl] = lax.shift_right_logical(idx_v[sl], 2)
            return carry

        lax.fori_loop(0, b_per_w // _LANES, shift_body, 0)

        copies = []
        for g in range(n_chunks):
            o = g * _IDX_CHUNK
            copies.append(
                pltpu.async_copy(
                    table_hbm.at[idx4_v.at[pl.ds(o, _IDX_CHUNK)]],
                    rows4_v.at[pl.ds(o, _IDX_CHUNK)],
                    sem,
                )
            )
        for c in copies:
            c.wait()

        lanes = jnp.arange(_LANES, dtype=jnp.int32)

        def ex_body(t, carry):
            sl = pl.ds(t * _LANES, _LANES)
            jvec = t * _LANES + lanes
            col = jnp.bitwise_and(idx_v[sl], rps - 1) * D
            dst = jvec * D
            for d in range(D):
                val = plsc.load_gather(rows4_v, [jvec, col + d])
                plsc.store_scatter(out_v, [dst + d], val)
            return carry

        lax.fori_loop(0, b_per_w // _LANES, ex_body, 0)

        pltpu.sync_copy(out_v, out_hbm.at[pl.ds(base * D, b_per_w * D)])

    return _gather(table4, colony_ids).reshape(B, D)
